# Initial kernel scaffold; baseline (speedup 1.0000x reference)
import jax
import jax.numpy as jnp
from jax.experimental import pallas as pl


def kernel(x, token_to_slot, lora_a, lora_b, lora_scaling):
    raise NotImplementedError("write your pallas kernel here")


# speedup vs baseline: 10.2511x; 10.2511x over previous
import functools

import jax
import jax.numpy as jnp
from jax.experimental import pallas as pl


_TB = 1024


def _lora_kernel(x_ref, slot_ref, a_ref, b_ref, o_ref, *, rank_shift):
    h = jnp.dot(x_ref[...], a_ref[...], preferred_element_type=jnp.float32)
    slot = slot_ref[0, 0, :]
    er = h.shape[1]
    eidx = jax.lax.broadcasted_iota(jnp.int32, (h.shape[0], er), 1) >> rank_shift
    hm = jnp.where(eidx == slot[:, None], h, 0.0)
    o_ref[...] = jnp.dot(hm, b_ref[...], preferred_element_type=jnp.float32)


def kernel(x, token_to_slot, lora_a, lora_b, lora_scaling):
    T, D = x.shape
    E, _, R = lora_a.shape
    D_OUT = lora_b.shape[-1]
    assert R & (R - 1) == 0
    rank_shift = R.bit_length() - 1

    a_all = jnp.transpose(lora_a, (1, 0, 2)).reshape(D, E * R)
    b_all = (lora_b * lora_scaling[:, None, None]).reshape(E * R, D_OUT)

    n_t = T // _TB
    slot3 = token_to_slot.reshape(n_t, 1, _TB)

    return pl.pallas_call(
        functools.partial(_lora_kernel, rank_shift=rank_shift),
        grid=(n_t,),
        in_specs=[
            pl.BlockSpec((_TB, D), lambda i: (i, 0)),
            pl.BlockSpec((1, 1, _TB), lambda i: (i, 0, 0)),
            pl.BlockSpec((D, E * R), lambda i: (0, 0)),
            pl.BlockSpec((E * R, D_OUT), lambda i: (0, 0)),
        ],
        out_specs=pl.BlockSpec((_TB, D_OUT), lambda i: (i, 0)),
        out_shape=jax.ShapeDtypeStruct((T, D_OUT), x.dtype),
    )(x, slot3, a_all, b_all)


# speedup vs baseline: 10.2840x; 1.0032x over previous
import functools

import jax
import jax.numpy as jnp
from jax.experimental import pallas as pl


_TB = 1024


def _lora_kernel(x_ref, slot_ref, a_ref, b_ref, o_ref, *, rank_shift):
    xb = x_ref[...].astype(jnp.bfloat16)
    h = jnp.dot(xb, a_ref[...], preferred_element_type=jnp.float32)
    slot = slot_ref[0, 0, :]
    er = h.shape[1]
    eidx = jax.lax.broadcasted_iota(jnp.int32, (h.shape[0], er), 1) >> rank_shift
    hm = jnp.where(eidx == slot[:, None], h, 0.0).astype(jnp.bfloat16)
    o_ref[...] = jnp.dot(hm, b_ref[...], preferred_element_type=jnp.float32)


def kernel(x, token_to_slot, lora_a, lora_b, lora_scaling):
    T, D = x.shape
    E, _, R = lora_a.shape
    D_OUT = lora_b.shape[-1]
    assert R & (R - 1) == 0
    rank_shift = R.bit_length() - 1

    a_all = jnp.transpose(lora_a, (1, 0, 2)).reshape(D, E * R).astype(jnp.bfloat16)
    b_all = (lora_b * lora_scaling[:, None, None]).reshape(E * R, D_OUT).astype(jnp.bfloat16)

    n_t = T // _TB
    slot3 = token_to_slot.reshape(n_t, 1, _TB)

    return pl.pallas_call(
        functools.partial(_lora_kernel, rank_shift=rank_shift),
        grid=(n_t,),
        in_specs=[
            pl.BlockSpec((_TB, D), lambda i: (i, 0)),
            pl.BlockSpec((1, 1, _TB), lambda i: (i, 0, 0)),
            pl.BlockSpec((D, E * R), lambda i: (0, 0)),
            pl.BlockSpec((E * R, D_OUT), lambda i: (0, 0)),
        ],
        out_specs=pl.BlockSpec((_TB, D_OUT), lambda i: (i, 0)),
        out_shape=jax.ShapeDtypeStruct((T, D_OUT), x.dtype),
    )(x, slot3, a_all, b_all)
